# baseline (speedup 1.0000x reference)
import functools

import numpy as np
import jax
import jax.numpy as jnp
from jax import lax
from jax.experimental import pallas as pl
from jax.experimental.pallas import tpu as pltpu
from jax.experimental.pallas import tpu_sc as plsc

N = 10000
E = 320000
D = 128
SCALE = 1.8

NC = 2
NS = 16
NW = NC * NS
CHUNK = 64
NCHT = 160
E_PAD = NW * NCHT * CHUNK
N_PAD = E_PAD - E
N_ZS = 10240
ZS_BN = 1024
ROWS_A = 632
ROWS_B = N - 15 * ROWS_A
N_DEG = 10240
DEG_PER_TILE = N_DEG // NS

_PAD_SRC = jnp.asarray(N + (np.arange(N_PAD, dtype=np.int32) % 240))
_PAD_DST = jnp.asarray(np.arange(N_PAD, dtype=np.int32))


@functools.lru_cache(maxsize=None)
def _sc_mesh():
    return plsc.VectorSubcoreMesh(
        core_axis_name="c", subcore_axis_name="s", num_cores=NC, num_subcores=NS
    )


def _deg_body(dst_hbm, out_hbm, didx, ones_v, zeros_v, dsem, isem, deg_sh):
    c = lax.axis_index("c")
    s = lax.axis_index("s")
    wid = c * NS + s
    e0 = wid * NCHT * CHUNK

    @pl.loop(0, CHUNK, step=16)
    def _(i):
        ones_v[pl.ds(i, 16)] = jnp.full((16,), 1.0, jnp.float32)

    @pl.loop(0, DEG_PER_TILE, step=16)
    def _(i):
        zeros_v[pl.ds(i, 16)] = jnp.zeros((16,), jnp.float32)

    pltpu.sync_copy(zeros_v, deg_sh.at[pl.ds(s * DEG_PER_TILE, DEG_PER_TILE)])

    def idx_load(j, slot):
        pltpu.async_copy(dst_hbm.at[pl.ds(e0 + j * CHUNK, CHUNK)],
                         didx.at[slot], isem.at[slot])

    def idx_wait(j, slot):
        pltpu.make_async_copy(dst_hbm.at[pl.ds(e0 + j * CHUNK, CHUNK)],
                              didx.at[slot], isem.at[slot]).wait()

    idx_load(0, 0)
    idx_load(1, 1)
    idx_load(2, 2)
    plsc.subcore_barrier()

    @pl.loop(0, NCHT, step=4)
    def _(i):
        for b in range(4):
            j = i + b
            idx_wait(j, b)
            pltpu.async_copy(ones_v, deg_sh.at[didx.at[b]], dsem, add=True)

            prev_slot = (b + 3) % 4
            if b == 0:
                @pl.when(j > 0)
                def _():
                    pltpu.make_async_copy(ones_v, deg_sh.at[didx.at[prev_slot]],
                                          dsem).wait()
            else:
                pltpu.make_async_copy(ones_v, deg_sh.at[didx.at[prev_slot]],
                                      dsem).wait()

            @pl.when(j + 3 < NCHT)
            def _():
                idx_load(j + 3, prev_slot)

    pltpu.make_async_copy(ones_v, deg_sh.at[didx.at[3]], dsem).wait()
    plsc.subcore_barrier()
    pltpu.sync_copy(
        deg_sh.at[pl.ds(s * DEG_PER_TILE, DEG_PER_TILE)],
        out_hbm.at[c, pl.ds(s * DEG_PER_TILE, DEG_PER_TILE)],
    )


@functools.lru_cache(maxsize=None)
def _deg_kernel():
    return pl.kernel(
        _deg_body,
        out_type=jax.ShapeDtypeStruct((NC, N_DEG), jnp.float32),
        mesh=_sc_mesh(),
        scratch_types=[
            pltpu.VMEM((4, CHUNK), jnp.int32),
            pltpu.VMEM((CHUNK,), jnp.float32),
            pltpu.VMEM((DEG_PER_TILE,), jnp.float32),
            pltpu.SemaphoreType.DMA,
            pltpu.SemaphoreType.DMA((4,)),
            pltpu.VMEM_SHARED((N_DEG,), jnp.float32),
        ],
    )


def _msg_body(zs_hbm, src_hbm, dst_hbm, out_hbm,
              sidx, didx, rows, gsem, ssem, isem, acc_sh):
    c = lax.axis_index("c")
    s = lax.axis_index("s")
    wid = c * NS + s
    r0 = s * ROWS_A
    e0 = wid * NCHT * CHUNK

    @pl.when(s < NS - 1)
    def _():
        pltpu.sync_copy(zs_hbm.at[pl.ds(r0, ROWS_A)],
                        acc_sh.at[pl.ds(r0, ROWS_A)])

    @pl.when(s == NS - 1)
    def _():
        pltpu.sync_copy(zs_hbm.at[pl.ds(15 * ROWS_A, ROWS_B)],
                        acc_sh.at[pl.ds(15 * ROWS_A, ROWS_B)])

    def idx_load(j, slot):
        pltpu.async_copy(src_hbm.at[pl.ds(e0 + j * CHUNK, CHUNK)],
                         sidx.at[slot], isem.at[slot])
        pltpu.async_copy(dst_hbm.at[pl.ds(e0 + j * CHUNK, CHUNK)],
                         didx.at[slot], isem.at[slot])

    def idx_wait(j, slot):
        pltpu.make_async_copy(src_hbm.at[pl.ds(e0 + j * CHUNK, CHUNK)],
                              sidx.at[slot], isem.at[slot]).wait()
        pltpu.make_async_copy(dst_hbm.at[pl.ds(e0 + j * CHUNK, CHUNK)],
                              didx.at[slot], isem.at[slot]).wait()

    def g_start(rslot, kslot):
        pltpu.async_copy(zs_hbm.at[sidx.at[kslot]], rows.at[rslot],
                         gsem.at[rslot])

    def g_wait(rslot, kslot):
        pltpu.make_async_copy(zs_hbm.at[sidx.at[kslot]], rows.at[rslot],
                              gsem.at[rslot]).wait()

    def s_start(rslot, kslot):
        pltpu.async_copy(rows.at[rslot], acc_sh.at[didx.at[kslot]],
                         ssem.at[rslot], add=True)

    def s_wait(rslot, kslot):
        pltpu.make_async_copy(rows.at[rslot], acc_sh.at[didx.at[kslot]],
                              ssem.at[rslot]).wait()

    pltpu.sync_copy(src_hbm.at[pl.ds(e0, CHUNK)], sidx.at[0])
    pltpu.sync_copy(dst_hbm.at[pl.ds(e0, CHUNK)], didx.at[0])
    g_start(0, 0)
    idx_load(1, 1)
    idx_load(2, 2)
    idx_load(3, 3)
    idx_load(4, 4)
    idx_wait(1, 1)
    g_start(1, 1)
    plsc.subcore_barrier()

    @pl.loop(0, NCHT, step=8)
    def _(i):
        for b in range(8):
            j = i + b
            r = b % 4
            k = b % 8

            g_wait(r, k)
            s_start(r, k)

            if b >= 2:
                s_wait((b - 2) % 4, (b - 2) % 8)
            else:
                @pl.when(j >= 2)
                def _():
                    s_wait((b - 2) % 4, (b - 2) % 8)

            if b < 6:
                @pl.when(j + 2 < NCHT)
                def _():
                    idx_wait(j + 2, (b + 2) % 8)
                    g_start((b + 2) % 4, (b + 2) % 8)
            else:
                @pl.when(j + 2 < NCHT)
                def _():
                    idx_wait(j + 2, (b + 2) % 8)
                    g_start((b + 2) % 4, (b + 2) % 8)

            @pl.when(j + 5 < NCHT)
            def _():
                idx_load(j + 5, (b + 5) % 8)

    s_wait((NCHT - 2) % 4, (NCHT - 2) % 8)
    s_wait((NCHT - 1) % 4, (NCHT - 1) % 8)
    plsc.subcore_barrier()

    @pl.when(s < NS - 1)
    def _():
        pltpu.sync_copy(acc_sh.at[pl.ds(r0, ROWS_A)],
                        out_hbm.at[c, pl.ds(r0, ROWS_A)])

    @pl.when(s == NS - 1)
    def _():
        pltpu.sync_copy(acc_sh.at[pl.ds(15 * ROWS_A, ROWS_B)],
                        out_hbm.at[c, pl.ds(15 * ROWS_A, ROWS_B)])


@functools.lru_cache(maxsize=None)
def _msg_kernel():
    return pl.kernel(
        _msg_body,
        out_type=jax.ShapeDtypeStruct((NC, N, D), jnp.float32),
        mesh=_sc_mesh(),
        scratch_types=[
            pltpu.VMEM((8, CHUNK), jnp.int32),
            pltpu.VMEM((8, CHUNK), jnp.int32),
            pltpu.VMEM((4, CHUNK, D), jnp.float32),
            pltpu.SemaphoreType.DMA((4,)),
            pltpu.SemaphoreType.DMA((4,)),
            pltpu.SemaphoreType.DMA((8,)),
            pltpu.VMEM_SHARED((N, D), jnp.float32),
        ],
    )


def _proj_body(x_ref, w_ref, b_ref, zn_ref):
    z = lax.dot_general(
        x_ref[...], w_ref[...], (((1,), (1,)), ((), ())),
        preferred_element_type=jnp.float32,
        precision=lax.Precision.HIGHEST,
    ) + b_ref[...]
    nrm = jnp.sqrt(jnp.sum(z * z, axis=1, keepdims=True))
    zn_ref[...] = z * (SCALE / jnp.maximum(nrm, 1e-12))


def _proj(x, W, b2, bn=1000):
    return pl.pallas_call(
        _proj_body,
        grid=(N // bn,),
        in_specs=[
            pl.BlockSpec((bn, D), lambda i: (i, 0)),
            pl.BlockSpec((D, D), lambda i: (0, 0)),
            pl.BlockSpec((1, D), lambda i: (0, 0)),
        ],
        out_specs=pl.BlockSpec((bn, D), lambda i: (i, 0)),
        out_shape=jax.ShapeDtypeStruct((N, D), jnp.float32),
    )(x, W, b2)


def _scale_body(zn_ref, ds_ref, zs_ref, dinv_ref):
    i = pl.program_id(0)
    rows = lax.broadcasted_iota(jnp.int32, (ZS_BN, 1), 0) + i * ZS_BN
    real = rows < N
    deg = ds_ref[...] + 1.0 - jnp.where(rows < N_PAD, 1.0, 0.0)
    dinv = jnp.where(real, lax.rsqrt(deg), 0.0)
    zs_ref[...] = jnp.where(real, zn_ref[...] * dinv, 0.0)
    dinv_ref[...] = dinv


def _scale(zn, dsum):
    return pl.pallas_call(
        _scale_body,
        grid=(N_ZS // ZS_BN,),
        in_specs=[
            pl.BlockSpec((ZS_BN, D), lambda i: (i, 0)),
            pl.BlockSpec((ZS_BN, 1), lambda i: (i, 0)),
        ],
        out_specs=[
            pl.BlockSpec((ZS_BN, D), lambda i: (i, 0)),
            pl.BlockSpec((ZS_BN, 1), lambda i: (i, 0)),
        ],
        out_shape=[
            jax.ShapeDtypeStruct((N_ZS, D), jnp.float32),
            jax.ShapeDtypeStruct((N_ZS, 1), jnp.float32),
        ],
    )(zn, dsum)


def _comb_body(p_ref, zs_ref, dinv_ref, out_ref):
    out_ref[...] = dinv_ref[...] * (p_ref[0] + p_ref[1] - zs_ref[...])


def _combine(parts, zs, dinv, bn=ZS_BN):
    return pl.pallas_call(
        _comb_body,
        grid=(N_ZS // bn,),
        in_specs=[
            pl.BlockSpec((NC, bn, D), lambda i: (0, i, 0)),
            pl.BlockSpec((bn, D), lambda i: (i, 0)),
            pl.BlockSpec((bn, 1), lambda i: (i, 0)),
        ],
        out_specs=pl.BlockSpec((bn, D), lambda i: (i, 0)),
        out_shape=jax.ShapeDtypeStruct((N, D), jnp.float32),
    )(parts, zs, dinv)


@jax.jit
def kernel(x, edge_index, W, b):
    src = edge_index[0].astype(jnp.int32)
    dst = edge_index[1].astype(jnp.int32)
    src1 = jnp.concatenate([src, _PAD_SRC])
    dst1 = jnp.concatenate([dst, _PAD_DST])

    deg_p = _deg_kernel()(dst1)
    zn = _proj(x, W, b.reshape(1, D))

    dsum = (deg_p[0] + deg_p[1])[:N].reshape(N, 1)
    zs, dinv = _scale(zn, dsum)

    parts = _msg_kernel()(zs, src1, dst1)
    return _combine(parts, zs, dinv)


# speedup vs baseline: 1.0474x; 1.0474x over previous
import functools

import numpy as np
import jax
import jax.numpy as jnp
from jax import lax
from jax.experimental import pallas as pl
from jax.experimental.pallas import tpu as pltpu
from jax.experimental.pallas import tpu_sc as plsc

N = 10000
E = 320000
D = 128
SCALE = 1.8

NC = 2
NS = 16
NW = NC * NS
CHUNK = 64
NCHT = 160
E_PAD = NW * NCHT * CHUNK
N_PAD = E_PAD - E
N_ZS = 10240
ZS_BN = 1024
ROWS_A = 632
ROWS_B = N - 15 * ROWS_A
N_DEG = 10240
DEG_PER_TILE = N_DEG // NS

EP_BN = 65536


@functools.lru_cache(maxsize=None)
def _sc_mesh():
    return plsc.VectorSubcoreMesh(
        core_axis_name="c", subcore_axis_name="s", num_cores=NC, num_subcores=NS
    )


def _deg_body(dst_hbm, out_hbm, didx, ones_v, zeros_v, dsem, isem, deg_sh):
    c = lax.axis_index("c")
    s = lax.axis_index("s")
    wid = c * NS + s
    e0 = wid * NCHT * CHUNK

    @pl.loop(0, CHUNK, step=16)
    def _(i):
        ones_v[pl.ds(i, 16)] = jnp.full((16,), 1.0, jnp.float32)

    @pl.loop(0, DEG_PER_TILE, step=16)
    def _(i):
        zeros_v[pl.ds(i, 16)] = jnp.zeros((16,), jnp.float32)

    pltpu.sync_copy(zeros_v, deg_sh.at[pl.ds(s * DEG_PER_TILE, DEG_PER_TILE)])

    def idx_load(j, slot):
        pltpu.async_copy(dst_hbm.at[pl.ds(e0 + j * CHUNK, CHUNK)],
                         didx.at[slot], isem.at[slot])

    def idx_wait(j, slot):
        pltpu.make_async_copy(dst_hbm.at[pl.ds(e0 + j * CHUNK, CHUNK)],
                              didx.at[slot], isem.at[slot]).wait()

    idx_load(0, 0)
    idx_load(1, 1)
    idx_load(2, 2)
    plsc.subcore_barrier()

    @pl.loop(0, NCHT, step=4)
    def _(i):
        for b in range(4):
            j = i + b
            idx_wait(j, b)
            pltpu.async_copy(ones_v, deg_sh.at[didx.at[b]], dsem, add=True)

            prev_slot = (b + 3) % 4
            if b == 0:
                @pl.when(j > 0)
                def _():
                    pltpu.make_async_copy(ones_v, deg_sh.at[didx.at[prev_slot]],
                                          dsem).wait()
            else:
                pltpu.make_async_copy(ones_v, deg_sh.at[didx.at[prev_slot]],
                                      dsem).wait()

            @pl.when(j + 3 < NCHT)
            def _():
                idx_load(j + 3, prev_slot)

    pltpu.make_async_copy(ones_v, deg_sh.at[didx.at[3]], dsem).wait()
    plsc.subcore_barrier()
    pltpu.sync_copy(
        deg_sh.at[pl.ds(s * DEG_PER_TILE, DEG_PER_TILE)],
        out_hbm.at[c, pl.ds(s * DEG_PER_TILE, DEG_PER_TILE)],
    )


@functools.lru_cache(maxsize=None)
def _deg_kernel():
    return pl.kernel(
        _deg_body,
        out_type=jax.ShapeDtypeStruct((NC, N_DEG), jnp.float32),
        mesh=_sc_mesh(),
        scratch_types=[
            pltpu.VMEM((4, CHUNK), jnp.int32),
            pltpu.VMEM((CHUNK,), jnp.float32),
            pltpu.VMEM((DEG_PER_TILE,), jnp.float32),
            pltpu.SemaphoreType.DMA,
            pltpu.SemaphoreType.DMA((4,)),
            pltpu.VMEM_SHARED((N_DEG,), jnp.float32),
        ],
    )


def _msg_body(zs_hbm, src_hbm, dst_hbm, out_hbm,
              sidx, didx, rows, gsem, ssem, isem, acc_sh):
    c = lax.axis_index("c")
    s = lax.axis_index("s")
    wid = c * NS + s
    r0 = s * ROWS_A
    e0 = wid * NCHT * CHUNK

    @pl.when(s < NS - 1)
    def _():
        pltpu.sync_copy(zs_hbm.at[pl.ds(r0, ROWS_A)],
                        acc_sh.at[pl.ds(r0, ROWS_A)])

    @pl.when(s == NS - 1)
    def _():
        pltpu.sync_copy(zs_hbm.at[pl.ds(15 * ROWS_A, ROWS_B)],
                        acc_sh.at[pl.ds(15 * ROWS_A, ROWS_B)])

    def idx_load(j, slot):
        pltpu.async_copy(src_hbm.at[pl.ds(e0 + j * CHUNK, CHUNK)],
                         sidx.at[slot], isem.at[slot])
        pltpu.async_copy(dst_hbm.at[pl.ds(e0 + j * CHUNK, CHUNK)],
                         didx.at[slot], isem.at[slot])

    def idx_wait(j, slot):
        pltpu.make_async_copy(src_hbm.at[pl.ds(e0 + j * CHUNK, CHUNK)],
                              sidx.at[slot], isem.at[slot]).wait()
        pltpu.make_async_copy(dst_hbm.at[pl.ds(e0 + j * CHUNK, CHUNK)],
                              didx.at[slot], isem.at[slot]).wait()

    def g_start(rslot, kslot):
        pltpu.async_copy(zs_hbm.at[sidx.at[kslot]], rows.at[rslot],
                         gsem.at[rslot])

    def g_wait(rslot, kslot):
        pltpu.make_async_copy(zs_hbm.at[sidx.at[kslot]], rows.at[rslot],
                              gsem.at[rslot]).wait()

    def s_start(rslot, kslot):
        pltpu.async_copy(rows.at[rslot], acc_sh.at[didx.at[kslot]],
                         ssem.at[rslot], add=True)

    def s_wait(rslot, kslot):
        pltpu.make_async_copy(rows.at[rslot], acc_sh.at[didx.at[kslot]],
                              ssem.at[rslot]).wait()

    pltpu.sync_copy(src_hbm.at[pl.ds(e0, CHUNK)], sidx.at[0])
    pltpu.sync_copy(dst_hbm.at[pl.ds(e0, CHUNK)], didx.at[0])
    g_start(0, 0)
    idx_load(1, 1)
    idx_load(2, 2)
    idx_load(3, 3)
    idx_load(4, 4)
    idx_wait(1, 1)
    g_start(1, 1)
    plsc.subcore_barrier()

    @pl.loop(0, NCHT, step=8)
    def _(i):
        for b in range(8):
            j = i + b
            r = b % 4
            k = b % 8

            g_wait(r, k)
            s_start(r, k)

            if b >= 2:
                s_wait((b - 2) % 4, (b - 2) % 8)
            else:
                @pl.when(j >= 2)
                def _():
                    s_wait((b - 2) % 4, (b - 2) % 8)

            if b < 6:
                @pl.when(j + 2 < NCHT)
                def _():
                    idx_wait(j + 2, (b + 2) % 8)
                    g_start((b + 2) % 4, (b + 2) % 8)
            else:
                @pl.when(j + 2 < NCHT)
                def _():
                    idx_wait(j + 2, (b + 2) % 8)
                    g_start((b + 2) % 4, (b + 2) % 8)

            @pl.when(j + 5 < NCHT)
            def _():
                idx_load(j + 5, (b + 5) % 8)

    s_wait((NCHT - 2) % 4, (NCHT - 2) % 8)
    s_wait((NCHT - 1) % 4, (NCHT - 1) % 8)
    plsc.subcore_barrier()

    @pl.when(s < NS - 1)
    def _():
        pltpu.sync_copy(acc_sh.at[pl.ds(r0, ROWS_A)],
                        out_hbm.at[c, pl.ds(r0, ROWS_A)])

    @pl.when(s == NS - 1)
    def _():
        pltpu.sync_copy(acc_sh.at[pl.ds(15 * ROWS_A, ROWS_B)],
                        out_hbm.at[c, pl.ds(15 * ROWS_A, ROWS_B)])


@functools.lru_cache(maxsize=None)
def _msg_kernel():
    return pl.kernel(
        _msg_body,
        out_type=jax.ShapeDtypeStruct((NC, N, D), jnp.float32),
        mesh=_sc_mesh(),
        scratch_types=[
            pltpu.VMEM((8, CHUNK), jnp.int32),
            pltpu.VMEM((8, CHUNK), jnp.int32),
            pltpu.VMEM((4, CHUNK, D), jnp.float32),
            pltpu.SemaphoreType.DMA((4,)),
            pltpu.SemaphoreType.DMA((4,)),
            pltpu.SemaphoreType.DMA((8,)),
            pltpu.VMEM_SHARED((N, D), jnp.float32),
        ],
    )


def _eprep_body(ei_ref, src_ref, dst_ref):
    i = pl.program_id(0)
    col = lax.broadcasted_iota(jnp.int32, (1, EP_BN), 1) + i * EP_BN
    real = col < E
    pcol = col - E
    src_ref[...] = jnp.where(real, ei_ref[0:1, :], N + (pcol & 127))[0]
    dst_ref[...] = jnp.where(real, ei_ref[1:2, :], pcol)[0]


def _eprep(ei):
    return pl.pallas_call(
        _eprep_body,
        grid=(E_PAD // EP_BN,),
        in_specs=[pl.BlockSpec((2, EP_BN), lambda i: (0, i))],
        out_specs=[
            pl.BlockSpec((EP_BN,), lambda i: (i,)),
            pl.BlockSpec((EP_BN,), lambda i: (i,)),
        ],
        out_shape=[
            jax.ShapeDtypeStruct((E_PAD,), jnp.int32),
            jax.ShapeDtypeStruct((E_PAD,), jnp.int32),
        ],
    )(ei)


def _proj_body(x_ref, w_ref, b_ref, zn_ref):
    z = lax.dot_general(
        x_ref[...], w_ref[...], (((1,), (1,)), ((), ())),
        preferred_element_type=jnp.float32,
        precision=lax.Precision.HIGHEST,
    ) + b_ref[...]
    nrm = jnp.sqrt(jnp.sum(z * z, axis=1, keepdims=True))
    zn_ref[...] = z * (SCALE / jnp.maximum(nrm, 1e-12))


def _proj(x, W, b2, bn=1000):
    return pl.pallas_call(
        _proj_body,
        grid=(N // bn,),
        in_specs=[
            pl.BlockSpec((bn, D), lambda i: (i, 0)),
            pl.BlockSpec((D, D), lambda i: (0, 0)),
            pl.BlockSpec((1, D), lambda i: (0, 0)),
        ],
        out_specs=pl.BlockSpec((bn, D), lambda i: (i, 0)),
        out_shape=jax.ShapeDtypeStruct((N, D), jnp.float32),
    )(x, W, b2)


def _scale_body(zn_ref, ds_ref, zs_ref, dinv_ref):
    i = pl.program_id(0)
    rows = lax.broadcasted_iota(jnp.int32, (ZS_BN, 1), 0) + i * ZS_BN
    real = rows < N
    deg = ds_ref[...] + 1.0 - jnp.where(rows < N_PAD, 1.0, 0.0)
    dinv = jnp.where(real, lax.rsqrt(deg), 0.0)
    zs_ref[...] = jnp.where(real, zn_ref[...] * dinv, 0.0)
    dinv_ref[...] = dinv


def _scale(zn, dsum):
    return pl.pallas_call(
        _scale_body,
        grid=(N_ZS // ZS_BN,),
        in_specs=[
            pl.BlockSpec((ZS_BN, D), lambda i: (i, 0)),
            pl.BlockSpec((ZS_BN, 1), lambda i: (i, 0)),
        ],
        out_specs=[
            pl.BlockSpec((ZS_BN, D), lambda i: (i, 0)),
            pl.BlockSpec((ZS_BN, 1), lambda i: (i, 0)),
        ],
        out_shape=[
            jax.ShapeDtypeStruct((N_ZS, D), jnp.float32),
            jax.ShapeDtypeStruct((N_ZS, 1), jnp.float32),
        ],
    )(zn, dsum)


def _comb_body(p_ref, zs_ref, dinv_ref, out_ref):
    out_ref[...] = dinv_ref[...] * (p_ref[0] + p_ref[1] - zs_ref[...])


def _combine(parts, zs, dinv, bn=ZS_BN):
    return pl.pallas_call(
        _comb_body,
        grid=(N_ZS // bn,),
        in_specs=[
            pl.BlockSpec((NC, bn, D), lambda i: (0, i, 0)),
            pl.BlockSpec((bn, D), lambda i: (i, 0)),
            pl.BlockSpec((bn, 1), lambda i: (i, 0)),
        ],
        out_specs=pl.BlockSpec((bn, D), lambda i: (i, 0)),
        out_shape=jax.ShapeDtypeStruct((N, D), jnp.float32),
    )(parts, zs, dinv)


@jax.jit
def kernel(x, edge_index, W, b):
    src1, dst1 = _eprep(edge_index.astype(jnp.int32))

    deg_p = _deg_kernel()(dst1)
    zn = _proj(x, W, b.reshape(1, D))

    dsum = (deg_p[0] + deg_p[1])[:N].reshape(N, 1)
    zs, dinv = _scale(zn, dsum)

    parts = _msg_kernel()(zs, src1, dst1)
    return _combine(parts, zs, dinv)


# speedup vs baseline: 1.0983x; 1.0486x over previous
import functools

import numpy as np
import jax
import jax.numpy as jnp
from jax import lax
from jax.experimental import pallas as pl
from jax.experimental.pallas import tpu as pltpu
from jax.experimental.pallas import tpu_sc as plsc

N = 10000
E = 320000
D = 128
SCALE = 1.8

NC = 2
NS = 16
NW = NC * NS
CHUNK = 64
NCHT = 160
DCH = 128
DNCHT = 80
E_PAD = NW * NCHT * CHUNK
N_PAD = E_PAD - E
N_ZS = 10240
ZS_BN = 1024
ROWS_A = 632
ROWS_B = N - 15 * ROWS_A
N_DEG = 10240
DEG_PER_TILE = N_DEG // NS

EP_BN = 65536


@functools.lru_cache(maxsize=None)
def _sc_mesh():
    return plsc.VectorSubcoreMesh(
        core_axis_name="c", subcore_axis_name="s", num_cores=NC, num_subcores=NS
    )


def _deg_body(dst_hbm, out_hbm, didx, ones_v, zeros_v, dsem, isem, deg_sh):
    c = lax.axis_index("c")
    s = lax.axis_index("s")
    wid = c * NS + s
    e0 = wid * DNCHT * DCH

    @pl.loop(0, DCH, step=16)
    def _(i):
        ones_v[pl.ds(i, 16)] = jnp.full((16,), 1.0, jnp.float32)

    @pl.loop(0, DEG_PER_TILE, step=16)
    def _(i):
        zeros_v[pl.ds(i, 16)] = jnp.zeros((16,), jnp.float32)

    pltpu.sync_copy(zeros_v, deg_sh.at[pl.ds(s * DEG_PER_TILE, DEG_PER_TILE)])

    def idx_load(j, slot):
        pltpu.async_copy(dst_hbm.at[pl.ds(e0 + j * DCH, DCH)],
                         didx.at[slot], isem.at[slot])

    def idx_wait(j, slot):
        pltpu.make_async_copy(dst_hbm.at[pl.ds(e0 + j * DCH, DCH)],
                              didx.at[slot], isem.at[slot]).wait()

    idx_load(0, 0)
    idx_load(1, 1)
    idx_load(2, 2)
    plsc.subcore_barrier()

    @pl.loop(0, DNCHT, step=4)
    def _(i):
        for b in range(4):
            j = i + b
            idx_wait(j, b)
            pltpu.async_copy(ones_v, deg_sh.at[didx.at[b]], dsem, add=True)

            prev_slot = (b + 3) % 4
            if b == 0:
                @pl.when(j > 0)
                def _():
                    pltpu.make_async_copy(ones_v, deg_sh.at[didx.at[prev_slot]],
                                          dsem).wait()
            else:
                pltpu.make_async_copy(ones_v, deg_sh.at[didx.at[prev_slot]],
                                      dsem).wait()

            @pl.when(j + 3 < DNCHT)
            def _():
                idx_load(j + 3, prev_slot)

    pltpu.make_async_copy(ones_v, deg_sh.at[didx.at[3]], dsem).wait()
    plsc.subcore_barrier()
    pltpu.sync_copy(
        deg_sh.at[pl.ds(s * DEG_PER_TILE, DEG_PER_TILE)],
        out_hbm.at[c, pl.ds(s * DEG_PER_TILE, DEG_PER_TILE)],
    )


@functools.lru_cache(maxsize=None)
def _deg_kernel():
    return pl.kernel(
        _deg_body,
        out_type=jax.ShapeDtypeStruct((NC, N_DEG), jnp.float32),
        mesh=_sc_mesh(),
        scratch_types=[
            pltpu.VMEM((4, DCH), jnp.int32),
            pltpu.VMEM((DCH,), jnp.float32),
            pltpu.VMEM((DEG_PER_TILE,), jnp.float32),
            pltpu.SemaphoreType.DMA,
            pltpu.SemaphoreType.DMA((4,)),
            pltpu.VMEM_SHARED((N_DEG,), jnp.float32),
        ],
    )


def _msg_body(zs_hbm, src_hbm, dst_hbm, out_hbm,
              sidx, didx, rows, gsem, ssem, isem, acc_sh):
    c = lax.axis_index("c")
    s = lax.axis_index("s")
    wid = c * NS + s
    r0 = s * ROWS_A
    e0 = wid * NCHT * CHUNK

    @pl.when(s < NS - 1)
    def _():
        pltpu.sync_copy(zs_hbm.at[pl.ds(r0, ROWS_A)],
                        acc_sh.at[pl.ds(r0, ROWS_A)])

    @pl.when(s == NS - 1)
    def _():
        pltpu.sync_copy(zs_hbm.at[pl.ds(15 * ROWS_A, ROWS_B)],
                        acc_sh.at[pl.ds(15 * ROWS_A, ROWS_B)])

    def idx_load(j, slot):
        pltpu.async_copy(src_hbm.at[pl.ds(e0 + j * CHUNK, CHUNK)],
                         sidx.at[slot], isem.at[slot])
        pltpu.async_copy(dst_hbm.at[pl.ds(e0 + j * CHUNK, CHUNK)],
                         didx.at[slot], isem.at[slot])

    def idx_wait(j, slot):
        pltpu.make_async_copy(src_hbm.at[pl.ds(e0 + j * CHUNK, CHUNK)],
                              sidx.at[slot], isem.at[slot]).wait()
        pltpu.make_async_copy(dst_hbm.at[pl.ds(e0 + j * CHUNK, CHUNK)],
                              didx.at[slot], isem.at[slot]).wait()

    def g_start(rslot, kslot):
        pltpu.async_copy(zs_hbm.at[sidx.at[kslot]], rows.at[rslot],
                         gsem.at[rslot])

    def g_wait(rslot, kslot):
        pltpu.make_async_copy(zs_hbm.at[sidx.at[kslot]], rows.at[rslot],
                              gsem.at[rslot]).wait()

    def s_start(rslot, kslot):
        pltpu.async_copy(rows.at[rslot], acc_sh.at[didx.at[kslot]],
                         ssem.at[rslot], add=True)

    def s_wait(rslot, kslot):
        pltpu.make_async_copy(rows.at[rslot], acc_sh.at[didx.at[kslot]],
                              ssem.at[rslot]).wait()

    pltpu.sync_copy(src_hbm.at[pl.ds(e0, CHUNK)], sidx.at[0])
    pltpu.sync_copy(dst_hbm.at[pl.ds(e0, CHUNK)], didx.at[0])
    g_start(0, 0)
    idx_load(1, 1)
    idx_load(2, 2)
    idx_load(3, 3)
    idx_load(4, 4)
    idx_wait(1, 1)
    g_start(1, 1)
    plsc.subcore_barrier()

    @pl.loop(0, NCHT, step=8)
    def _(i):
        for b in range(8):
            j = i + b
            r = b % 4
            k = b % 8

            g_wait(r, k)
            s_start(r, k)

            if b >= 2:
                s_wait((b - 2) % 4, (b - 2) % 8)
            else:
                @pl.when(j >= 2)
                def _():
                    s_wait((b - 2) % 4, (b - 2) % 8)

            if b < 6:
                @pl.when(j + 2 < NCHT)
                def _():
                    idx_wait(j + 2, (b + 2) % 8)
                    g_start((b + 2) % 4, (b + 2) % 8)
            else:
                @pl.when(j + 2 < NCHT)
                def _():
                    idx_wait(j + 2, (b + 2) % 8)
                    g_start((b + 2) % 4, (b + 2) % 8)

            @pl.when(j + 5 < NCHT)
            def _():
                idx_load(j + 5, (b + 5) % 8)

    s_wait((NCHT - 2) % 4, (NCHT - 2) % 8)
    s_wait((NCHT - 1) % 4, (NCHT - 1) % 8)
    plsc.subcore_barrier()

    @pl.when(s < NS - 1)
    def _():
        pltpu.sync_copy(acc_sh.at[pl.ds(r0, ROWS_A)],
                        out_hbm.at[c, pl.ds(r0, ROWS_A)])

    @pl.when(s == NS - 1)
    def _():
        pltpu.sync_copy(acc_sh.at[pl.ds(15 * ROWS_A, ROWS_B)],
                        out_hbm.at[c, pl.ds(15 * ROWS_A, ROWS_B)])


@functools.lru_cache(maxsize=None)
def _msg_kernel():
    return pl.kernel(
        _msg_body,
        out_type=jax.ShapeDtypeStruct((NC, N, D), jnp.float32),
        mesh=_sc_mesh(),
        scratch_types=[
            pltpu.VMEM((8, CHUNK), jnp.int32),
            pltpu.VMEM((8, CHUNK), jnp.int32),
            pltpu.VMEM((4, CHUNK, D), jnp.float32),
            pltpu.SemaphoreType.DMA((4,)),
            pltpu.SemaphoreType.DMA((4,)),
            pltpu.SemaphoreType.DMA((8,)),
            pltpu.VMEM_SHARED((N, D), jnp.float32),
        ],
    )


def _eprep_body(ei_ref, src_ref, dst_ref):
    i = pl.program_id(0)
    col = lax.broadcasted_iota(jnp.int32, (1, EP_BN), 1) + i * EP_BN
    real = col < E
    pcol = col - E
    src_ref[...] = jnp.where(real, ei_ref[0:1, :], N + (pcol & 127))[0]
    dst_ref[...] = jnp.where(real, ei_ref[1:2, :], pcol)[0]


def _eprep(ei):
    return pl.pallas_call(
        _eprep_body,
        grid=(E_PAD // EP_BN,),
        in_specs=[pl.BlockSpec((2, EP_BN), lambda i: (0, i))],
        out_specs=[
            pl.BlockSpec((EP_BN,), lambda i: (i,)),
            pl.BlockSpec((EP_BN,), lambda i: (i,)),
        ],
        out_shape=[
            jax.ShapeDtypeStruct((E_PAD,), jnp.int32),
            jax.ShapeDtypeStruct((E_PAD,), jnp.int32),
        ],
    )(ei)


def _proj_body(x_ref, w_ref, b_ref, zn_ref):
    z = lax.dot_general(
        x_ref[...], w_ref[...], (((1,), (1,)), ((), ())),
        preferred_element_type=jnp.float32,
        precision=lax.Precision.DEFAULT,
    ) + b_ref[...]
    nrm = jnp.sqrt(jnp.sum(z * z, axis=1, keepdims=True))
    zn_ref[...] = z * (SCALE / jnp.maximum(nrm, 1e-12))


def _proj(x, W, b2, bn=1000):
    return pl.pallas_call(
        _proj_body,
        grid=(N // bn,),
        in_specs=[
            pl.BlockSpec((bn, D), lambda i: (i, 0)),
            pl.BlockSpec((D, D), lambda i: (0, 0)),
            pl.BlockSpec((1, D), lambda i: (0, 0)),
        ],
        out_specs=pl.BlockSpec((bn, D), lambda i: (i, 0)),
        out_shape=jax.ShapeDtypeStruct((N, D), jnp.float32),
    )(x, W, b2)


def _scale_body(zn_ref, ds_ref, zs_ref, dinv_ref):
    i = pl.program_id(0)
    rows = lax.broadcasted_iota(jnp.int32, (ZS_BN, 1), 0) + i * ZS_BN
    real = rows < N
    deg = ds_ref[...] + 1.0 - jnp.where(rows < N_PAD, 1.0, 0.0)
    dinv = jnp.where(real, lax.rsqrt(deg), 0.0)
    zs_ref[...] = jnp.where(real, zn_ref[...] * dinv, 0.0)
    dinv_ref[...] = dinv


def _scale(zn, dsum):
    return pl.pallas_call(
        _scale_body,
        grid=(N_ZS // ZS_BN,),
        in_specs=[
            pl.BlockSpec((ZS_BN, D), lambda i: (i, 0)),
            pl.BlockSpec((ZS_BN, 1), lambda i: (i, 0)),
        ],
        out_specs=[
            pl.BlockSpec((ZS_BN, D), lambda i: (i, 0)),
            pl.BlockSpec((ZS_BN, 1), lambda i: (i, 0)),
        ],
        out_shape=[
            jax.ShapeDtypeStruct((N_ZS, D), jnp.float32),
            jax.ShapeDtypeStruct((N_ZS, 1), jnp.float32),
        ],
    )(zn, dsum)


def _comb_body(p_ref, zs_ref, dinv_ref, out_ref):
    out_ref[...] = dinv_ref[...] * (p_ref[0] + p_ref[1] - zs_ref[...])


def _combine(parts, zs, dinv, bn=ZS_BN):
    return pl.pallas_call(
        _comb_body,
        grid=(N_ZS // bn,),
        in_specs=[
            pl.BlockSpec((NC, bn, D), lambda i: (0, i, 0)),
            pl.BlockSpec((bn, D), lambda i: (i, 0)),
            pl.BlockSpec((bn, 1), lambda i: (i, 0)),
        ],
        out_specs=pl.BlockSpec((bn, D), lambda i: (i, 0)),
        out_shape=jax.ShapeDtypeStruct((N, D), jnp.float32),
    )(parts, zs, dinv)


@jax.jit
def kernel(x, edge_index, W, b):
    src1, dst1 = _eprep(edge_index.astype(jnp.int32))

    deg_p = _deg_kernel()(dst1)
    zn = _proj(x, W, b.reshape(1, D))

    dsum = (deg_p[0] + deg_p[1])[:N].reshape(N, 1)
    zs, dinv = _scale(zn, dsum)

    parts = _msg_kernel()(zs, src1, dst1)
    return _combine(parts, zs, dinv)


# speedup vs baseline: 1.1005x; 1.0020x over previous
import functools

import jax
import jax.numpy as jnp
from jax import lax
from jax.experimental import pallas as pl
from jax.experimental.pallas import tpu as pltpu
from jax.experimental.pallas import tpu_sc as plsc

N = 10000
E = 320000
D = 128
SCALE = 1.8

NC = 2
NS = 16
NW = NC * NS
CHUNK = 64
NCHT = 160
DCH = 128
DNCHT = 80
E_PAD = NW * NCHT * CHUNK
N_PAD = E_PAD - E
N_ZS = 10240
ZS_BN = 1024
ROWS_A = 632
ROWS_B = N - 15 * ROWS_A
N_DEG = 10240
DEG_PER_TILE = N_DEG // NS

EP_BN = 65536


@functools.lru_cache(maxsize=None)
def _sc_mesh():
    return plsc.VectorSubcoreMesh(
        core_axis_name="c", subcore_axis_name="s", num_cores=NC, num_subcores=NS
    )


def _deg_body(dst_hbm, out_hbm, didx, ones_v, zeros_v, dsem, isem, deg_sh):
    c = lax.axis_index("c")
    s = lax.axis_index("s")
    wid = c * NS + s
    e0 = wid * DNCHT * DCH

    @pl.loop(0, DCH, step=16)
    def _(i):
        ones_v[pl.ds(i, 16)] = jnp.full((16,), 1.0, jnp.float32)

    @pl.loop(0, DEG_PER_TILE, step=16)
    def _(i):
        zeros_v[pl.ds(i, 16)] = jnp.zeros((16,), jnp.float32)

    pltpu.sync_copy(zeros_v, deg_sh.at[pl.ds(s * DEG_PER_TILE, DEG_PER_TILE)])

    def idx_load(j, slot):
        pltpu.async_copy(dst_hbm.at[pl.ds(e0 + j * DCH, DCH)],
                         didx.at[slot], isem.at[slot])

    def idx_wait(j, slot):
        pltpu.make_async_copy(dst_hbm.at[pl.ds(e0 + j * DCH, DCH)],
                              didx.at[slot], isem.at[slot]).wait()

    idx_load(0, 0)
    idx_load(1, 1)
    idx_load(2, 2)
    plsc.subcore_barrier()

    @pl.loop(0, DNCHT, step=4)
    def _(i):
        for b in range(4):
            j = i + b
            idx_wait(j, b)
            pltpu.async_copy(ones_v, deg_sh.at[didx.at[b]], dsem, add=True)

            prev_slot = (b + 3) % 4
            if b == 0:
                @pl.when(j > 0)
                def _():
                    pltpu.make_async_copy(ones_v, deg_sh.at[didx.at[prev_slot]],
                                          dsem).wait()
            else:
                pltpu.make_async_copy(ones_v, deg_sh.at[didx.at[prev_slot]],
                                      dsem).wait()

            @pl.when(j + 3 < DNCHT)
            def _():
                idx_load(j + 3, prev_slot)

    pltpu.make_async_copy(ones_v, deg_sh.at[didx.at[3]], dsem).wait()
    plsc.subcore_barrier()
    pltpu.sync_copy(
        deg_sh.at[pl.ds(s * DEG_PER_TILE, DEG_PER_TILE)],
        out_hbm.at[c, pl.ds(s * DEG_PER_TILE, DEG_PER_TILE)],
    )


@functools.lru_cache(maxsize=None)
def _deg_kernel():
    return pl.kernel(
        _deg_body,
        out_type=jax.ShapeDtypeStruct((NC, N_DEG), jnp.float32),
        mesh=_sc_mesh(),
        scratch_types=[
            pltpu.VMEM((4, DCH), jnp.int32),
            pltpu.VMEM((DCH,), jnp.float32),
            pltpu.VMEM((DEG_PER_TILE,), jnp.float32),
            pltpu.SemaphoreType.DMA,
            pltpu.SemaphoreType.DMA((4,)),
            pltpu.VMEM_SHARED((N_DEG,), jnp.float32),
        ],
    )


def _msg_body(zs_hbm, src_hbm, dst_hbm, out_hbm,
              sidx, didx, rows, gsem, ssem, isem, acc_sh):
    c = lax.axis_index("c")
    s = lax.axis_index("s")
    wid = c * NS + s
    r0 = s * ROWS_A
    e0 = wid * NCHT * CHUNK

    @pl.when(s < NS - 1)
    def _():
        pltpu.sync_copy(zs_hbm.at[pl.ds(r0, ROWS_A)],
                        acc_sh.at[pl.ds(r0, ROWS_A)])

    @pl.when(s == NS - 1)
    def _():
        pltpu.sync_copy(zs_hbm.at[pl.ds(15 * ROWS_A, ROWS_B)],
                        acc_sh.at[pl.ds(15 * ROWS_A, ROWS_B)])

    def idx_load(j, slot):
        pltpu.async_copy(src_hbm.at[pl.ds(e0 + j * CHUNK, CHUNK)],
                         sidx.at[slot], isem.at[slot])
        pltpu.async_copy(dst_hbm.at[pl.ds(e0 + j * CHUNK, CHUNK)],
                         didx.at[slot], isem.at[slot])

    def idx_wait(j, slot):
        pltpu.make_async_copy(src_hbm.at[pl.ds(e0 + j * CHUNK, CHUNK)],
                              sidx.at[slot], isem.at[slot]).wait()
        pltpu.make_async_copy(dst_hbm.at[pl.ds(e0 + j * CHUNK, CHUNK)],
                              didx.at[slot], isem.at[slot]).wait()

    def g_start(rslot, kslot):
        pltpu.async_copy(zs_hbm.at[sidx.at[kslot]], rows.at[rslot],
                         gsem.at[rslot])

    def g_wait(rslot, kslot):
        pltpu.make_async_copy(zs_hbm.at[sidx.at[kslot]], rows.at[rslot],
                              gsem.at[rslot]).wait()

    def s_start(rslot, kslot):
        pltpu.async_copy(rows.at[rslot], acc_sh.at[didx.at[kslot]],
                         ssem.at[rslot], add=True)

    def s_wait(rslot, kslot):
        pltpu.make_async_copy(rows.at[rslot], acc_sh.at[didx.at[kslot]],
                              ssem.at[rslot]).wait()

    pltpu.sync_copy(src_hbm.at[pl.ds(e0, CHUNK)], sidx.at[0])
    pltpu.sync_copy(dst_hbm.at[pl.ds(e0, CHUNK)], didx.at[0])
    g_start(0, 0)
    idx_load(1, 1)
    idx_load(2, 2)
    idx_load(3, 3)
    idx_load(4, 4)
    idx_wait(1, 1)
    g_start(1, 1)
    plsc.subcore_barrier()

    @pl.loop(0, NCHT, step=8)
    def _(i):
        for b in range(8):
            j = i + b
            r = b % 4
            k = b % 8

            g_wait(r, k)
            s_start(r, k)

            if b >= 2:
                s_wait((b - 2) % 4, (b - 2) % 8)
            else:
                @pl.when(j >= 2)
                def _():
                    s_wait((b - 2) % 4, (b - 2) % 8)

            if b < 6:
                @pl.when(j + 2 < NCHT)
                def _():
                    idx_wait(j + 2, (b + 2) % 8)
                    g_start((b + 2) % 4, (b + 2) % 8)
            else:
                @pl.when(j + 2 < NCHT)
                def _():
                    idx_wait(j + 2, (b + 2) % 8)
                    g_start((b + 2) % 4, (b + 2) % 8)

            @pl.when(j + 5 < NCHT)
            def _():
                idx_load(j + 5, (b + 5) % 8)

    s_wait((NCHT - 2) % 4, (NCHT - 2) % 8)
    s_wait((NCHT - 1) % 4, (NCHT - 1) % 8)
    plsc.subcore_barrier()

    @pl.when(s < NS - 1)
    def _():
        pltpu.sync_copy(acc_sh.at[pl.ds(r0, ROWS_A)],
                        out_hbm.at[c, pl.ds(r0, ROWS_A)])

    @pl.when(s == NS - 1)
    def _():
        pltpu.sync_copy(acc_sh.at[pl.ds(15 * ROWS_A, ROWS_B)],
                        out_hbm.at[c, pl.ds(15 * ROWS_A, ROWS_B)])


@functools.lru_cache(maxsize=None)
def _msg_kernel():
    return pl.kernel(
        _msg_body,
        out_type=jax.ShapeDtypeStruct((NC, N, D), jnp.float32),
        mesh=_sc_mesh(),
        scratch_types=[
            pltpu.VMEM((8, CHUNK), jnp.int32),
            pltpu.VMEM((8, CHUNK), jnp.int32),
            pltpu.VMEM((4, CHUNK, D), jnp.float32),
            pltpu.SemaphoreType.DMA((4,)),
            pltpu.SemaphoreType.DMA((4,)),
            pltpu.SemaphoreType.DMA((8,)),
            pltpu.VMEM_SHARED((N, D), jnp.float32),
        ],
    )


def _eprep_body(ei_ref, src_ref, dst_ref):
    i = pl.program_id(0)
    col = lax.broadcasted_iota(jnp.int32, (1, EP_BN), 1) + i * EP_BN
    real = col < E
    pcol = col - E
    src_ref[...] = jnp.where(real, ei_ref[0:1, :], N + (pcol & 127))[0]
    dst_ref[...] = jnp.where(real, ei_ref[1:2, :], pcol)[0]


def _eprep(ei):
    return pl.pallas_call(
        _eprep_body,
        grid=(E_PAD // EP_BN,),
        in_specs=[pl.BlockSpec((2, EP_BN), lambda i: (0, i))],
        out_specs=[
            pl.BlockSpec((EP_BN,), lambda i: (i,)),
            pl.BlockSpec((EP_BN,), lambda i: (i,)),
        ],
        out_shape=[
            jax.ShapeDtypeStruct((E_PAD,), jnp.int32),
            jax.ShapeDtypeStruct((E_PAD,), jnp.int32),
        ],
    )(ei)


def _proj_body(x_ref, w_ref, b_ref, zn_ref):
    z = lax.dot_general(
        x_ref[...], w_ref[...], (((1,), (1,)), ((), ())),
        preferred_element_type=jnp.float32,
        precision=lax.Precision.DEFAULT,
    ) + b_ref[...]
    nrm = jnp.sqrt(jnp.sum(z * z, axis=1, keepdims=True))
    zn_ref[...] = z * (SCALE / jnp.maximum(nrm, 1e-12))


def _proj(x, W, b2, bn=1000):
    return pl.pallas_call(
        _proj_body,
        grid=(N // bn,),
        in_specs=[
            pl.BlockSpec((bn, D), lambda i: (i, 0)),
            pl.BlockSpec((D, D), lambda i: (0, 0)),
            pl.BlockSpec((1, D), lambda i: (0, 0)),
        ],
        out_specs=pl.BlockSpec((bn, D), lambda i: (i, 0)),
        out_shape=jax.ShapeDtypeStruct((N, D), jnp.float32),
    )(x, W, b2)


def _scale_body(zn_ref, ds_ref, zs_ref, dinv_ref):
    i = pl.program_id(0)
    rows = lax.broadcasted_iota(jnp.int32, (ZS_BN, 1), 0) + i * ZS_BN
    real = rows < N
    deg = ds_ref[...] + 1.0 - jnp.where(rows < N_PAD, 1.0, 0.0)
    dinv = jnp.where(real, lax.rsqrt(deg), 0.0)
    zs_ref[...] = jnp.where(real, zn_ref[...] * dinv, 0.0)
    dinv_ref[...] = dinv


def _scale(zn, dsum):
    return pl.pallas_call(
        _scale_body,
        grid=(N_ZS // ZS_BN,),
        in_specs=[
            pl.BlockSpec((ZS_BN, D), lambda i: (i, 0)),
            pl.BlockSpec((ZS_BN, 1), lambda i: (i, 0)),
        ],
        out_specs=[
            pl.BlockSpec((ZS_BN, D), lambda i: (i, 0)),
            pl.BlockSpec((ZS_BN, 1), lambda i: (i, 0)),
        ],
        out_shape=[
            jax.ShapeDtypeStruct((N_ZS, D), jnp.float32),
            jax.ShapeDtypeStruct((N_ZS, 1), jnp.float32),
        ],
    )(zn, dsum)


def _comb_body(p_ref, zs_ref, dinv_ref, out_ref):
    out_ref[...] = dinv_ref[...] * (p_ref[0] + p_ref[1] - zs_ref[...])


def _combine(parts, zs, dinv, bn=ZS_BN):
    return pl.pallas_call(
        _comb_body,
        grid=(N_ZS // bn,),
        in_specs=[
            pl.BlockSpec((NC, bn, D), lambda i: (0, i, 0)),
            pl.BlockSpec((bn, D), lambda i: (i, 0)),
            pl.BlockSpec((bn, 1), lambda i: (i, 0)),
        ],
        out_specs=pl.BlockSpec((bn, D), lambda i: (i, 0)),
        out_shape=jax.ShapeDtypeStruct((N, D), jnp.float32),
    )(parts, zs, dinv)


@jax.jit
def kernel(x, edge_index, W, b):
    src1, dst1 = _eprep(edge_index.astype(jnp.int32))

    deg_p = _deg_kernel()(dst1)
    zn = _proj(x, W, b.reshape(1, D))

    dsum = (deg_p[0] + deg_p[1])[:N].reshape(N, 1)
    zs, dinv = _scale(zn, dsum)

    parts = _msg_kernel()(zs, src1, dst1)
    return _combine(parts, zs, dinv)


# speedup vs baseline: 1.1268x; 1.0239x over previous
import functools

import jax
import jax.numpy as jnp
from jax import lax
from jax.experimental import pallas as pl
from jax.experimental.pallas import tpu as pltpu
from jax.experimental.pallas import tpu_sc as plsc

N = 10000
E = 320000
D = 128
SCALE = 1.8

NC = 2
NS = 16
NW = NC * NS
CHUNK = 64
NCHT = 160
DCH = 128
DNCHT = 80
E_PAD = NW * NCHT * CHUNK
N_PAD = E_PAD - E
N_ZS = 10240
ZS_BN = 2048
ROWS_A = 632
ROWS_B = N - 15 * ROWS_A
N_DEG = 10240
DEG_PER_TILE = N_DEG // NS

EP_BN = 65536


@functools.lru_cache(maxsize=None)
def _sc_mesh():
    return plsc.VectorSubcoreMesh(
        core_axis_name="c", subcore_axis_name="s", num_cores=NC, num_subcores=NS
    )


def _deg_body(dst_hbm, out_hbm, didx, ones_v, zeros_v, dsem, isem, deg_sh):
    c = lax.axis_index("c")
    s = lax.axis_index("s")
    wid = c * NS + s
    e0 = wid * DNCHT * DCH

    @pl.loop(0, DCH, step=16)
    def _(i):
        ones_v[pl.ds(i, 16)] = jnp.full((16,), 1.0, jnp.float32)

    @pl.loop(0, DEG_PER_TILE, step=16)
    def _(i):
        zeros_v[pl.ds(i, 16)] = jnp.zeros((16,), jnp.float32)

    pltpu.sync_copy(zeros_v, deg_sh.at[pl.ds(s * DEG_PER_TILE, DEG_PER_TILE)])

    def idx_load(j, slot):
        pltpu.async_copy(dst_hbm.at[pl.ds(e0 + j * DCH, DCH)],
                         didx.at[slot], isem.at[slot])

    def idx_wait(j, slot):
        pltpu.make_async_copy(dst_hbm.at[pl.ds(e0 + j * DCH, DCH)],
                              didx.at[slot], isem.at[slot]).wait()

    idx_load(0, 0)
    idx_load(1, 1)
    idx_load(2, 2)
    plsc.subcore_barrier()

    @pl.loop(0, DNCHT, step=4)
    def _(i):
        for b in range(4):
            j = i + b
            idx_wait(j, b)
            pltpu.async_copy(ones_v, deg_sh.at[didx.at[b]], dsem, add=True)

            prev_slot = (b + 3) % 4
            if b == 0:
                @pl.when(j > 0)
                def _():
                    pltpu.make_async_copy(ones_v, deg_sh.at[didx.at[prev_slot]],
                                          dsem).wait()
            else:
                pltpu.make_async_copy(ones_v, deg_sh.at[didx.at[prev_slot]],
                                      dsem).wait()

            @pl.when(j + 3 < DNCHT)
            def _():
                idx_load(j + 3, prev_slot)

    pltpu.make_async_copy(ones_v, deg_sh.at[didx.at[3]], dsem).wait()
    plsc.subcore_barrier()
    pltpu.sync_copy(
        deg_sh.at[pl.ds(s * DEG_PER_TILE, DEG_PER_TILE)],
        out_hbm.at[c, pl.ds(s * DEG_PER_TILE, DEG_PER_TILE)],
    )


@functools.lru_cache(maxsize=None)
def _deg_kernel():
    return pl.kernel(
        _deg_body,
        out_type=jax.ShapeDtypeStruct((NC, N_DEG), jnp.float32),
        mesh=_sc_mesh(),
        scratch_types=[
            pltpu.VMEM((4, DCH), jnp.int32),
            pltpu.VMEM((DCH,), jnp.float32),
            pltpu.VMEM((DEG_PER_TILE,), jnp.float32),
            pltpu.SemaphoreType.DMA,
            pltpu.SemaphoreType.DMA((4,)),
            pltpu.VMEM_SHARED((N_DEG,), jnp.float32),
        ],
    )


def _msg_body(zs_hbm, src_hbm, dst_hbm, out_hbm,
              sidx, didx, rows, gsem, ssem, isem, acc_sh):
    c = lax.axis_index("c")
    s = lax.axis_index("s")
    wid = c * NS + s
    r0 = s * ROWS_A
    e0 = wid * NCHT * CHUNK

    @pl.when(s < NS - 1)
    def _():
        pltpu.sync_copy(zs_hbm.at[pl.ds(r0, ROWS_A)],
                        acc_sh.at[pl.ds(r0, ROWS_A)])

    @pl.when(s == NS - 1)
    def _():
        pltpu.sync_copy(zs_hbm.at[pl.ds(15 * ROWS_A, ROWS_B)],
                        acc_sh.at[pl.ds(15 * ROWS_A, ROWS_B)])

    def idx_load(j, slot):
        pltpu.async_copy(src_hbm.at[pl.ds(e0 + j * CHUNK, CHUNK)],
                         sidx.at[slot], isem.at[slot])
        pltpu.async_copy(dst_hbm.at[pl.ds(e0 + j * CHUNK, CHUNK)],
                         didx.at[slot], isem.at[slot])

    def idx_wait(j, slot):
        pltpu.make_async_copy(src_hbm.at[pl.ds(e0 + j * CHUNK, CHUNK)],
                              sidx.at[slot], isem.at[slot]).wait()
        pltpu.make_async_copy(dst_hbm.at[pl.ds(e0 + j * CHUNK, CHUNK)],
                              didx.at[slot], isem.at[slot]).wait()

    def g_start(rslot, kslot):
        pltpu.async_copy(zs_hbm.at[sidx.at[kslot]], rows.at[rslot],
                         gsem.at[rslot])

    def g_wait(rslot, kslot):
        pltpu.make_async_copy(zs_hbm.at[sidx.at[kslot]], rows.at[rslot],
                              gsem.at[rslot]).wait()

    def s_start(rslot, kslot):
        pltpu.async_copy(rows.at[rslot], acc_sh.at[didx.at[kslot]],
                         ssem.at[rslot], add=True)

    def s_wait(rslot, kslot):
        pltpu.make_async_copy(rows.at[rslot], acc_sh.at[didx.at[kslot]],
                              ssem.at[rslot]).wait()

    pltpu.sync_copy(src_hbm.at[pl.ds(e0, CHUNK)], sidx.at[0])
    pltpu.sync_copy(dst_hbm.at[pl.ds(e0, CHUNK)], didx.at[0])
    g_start(0, 0)
    idx_load(1, 1)
    idx_load(2, 2)
    idx_load(3, 3)
    idx_load(4, 4)
    idx_wait(1, 1)
    g_start(1, 1)
    plsc.subcore_barrier()

    @pl.loop(0, NCHT, step=8)
    def _(i):
        for b in range(8):
            j = i + b
            r = b % 4
            k = b % 8

            g_wait(r, k)
            s_start(r, k)

            if b >= 2:
                s_wait((b - 2) % 4, (b - 2) % 8)
            else:
                @pl.when(j >= 2)
                def _():
                    s_wait((b - 2) % 4, (b - 2) % 8)

            if b < 6:
                @pl.when(j + 2 < NCHT)
                def _():
                    idx_wait(j + 2, (b + 2) % 8)
                    g_start((b + 2) % 4, (b + 2) % 8)
            else:
                @pl.when(j + 2 < NCHT)
                def _():
                    idx_wait(j + 2, (b + 2) % 8)
                    g_start((b + 2) % 4, (b + 2) % 8)

            @pl.when(j + 5 < NCHT)
            def _():
                idx_load(j + 5, (b + 5) % 8)

    s_wait((NCHT - 2) % 4, (NCHT - 2) % 8)
    s_wait((NCHT - 1) % 4, (NCHT - 1) % 8)
    plsc.subcore_barrier()

    @pl.when(s < NS - 1)
    def _():
        pltpu.sync_copy(acc_sh.at[pl.ds(r0, ROWS_A)],
                        out_hbm.at[c, pl.ds(r0, ROWS_A)])

    @pl.when(s == NS - 1)
    def _():
        pltpu.sync_copy(acc_sh.at[pl.ds(15 * ROWS_A, ROWS_B)],
                        out_hbm.at[c, pl.ds(15 * ROWS_A, ROWS_B)])


@functools.lru_cache(maxsize=None)
def _msg_kernel():
    return pl.kernel(
        _msg_body,
        out_type=jax.ShapeDtypeStruct((NC, N, D), jnp.float32),
        mesh=_sc_mesh(),
        scratch_types=[
            pltpu.VMEM((8, CHUNK), jnp.int32),
            pltpu.VMEM((8, CHUNK), jnp.int32),
            pltpu.VMEM((4, CHUNK, D), jnp.float32),
            pltpu.SemaphoreType.DMA((4,)),
            pltpu.SemaphoreType.DMA((4,)),
            pltpu.SemaphoreType.DMA((8,)),
            pltpu.VMEM_SHARED((N, D), jnp.float32),
        ],
    )


def _eprep_body(ei_ref, src_ref, dst_ref):
    i = pl.program_id(0)
    col = lax.broadcasted_iota(jnp.int32, (1, EP_BN), 1) + i * EP_BN
    real = col < E
    pcol = col - E
    src_ref[...] = jnp.where(real, ei_ref[0:1, :], N + (pcol & 127))[0]
    dst_ref[...] = jnp.where(real, ei_ref[1:2, :], pcol)[0]


def _eprep(ei):
    return pl.pallas_call(
        _eprep_body,
        grid=(E_PAD // EP_BN,),
        in_specs=[pl.BlockSpec((2, EP_BN), lambda i: (0, i))],
        out_specs=[
            pl.BlockSpec((EP_BN,), lambda i: (i,)),
            pl.BlockSpec((EP_BN,), lambda i: (i,)),
        ],
        out_shape=[
            jax.ShapeDtypeStruct((E_PAD,), jnp.int32),
            jax.ShapeDtypeStruct((E_PAD,), jnp.int32),
        ],
    )(ei)


def _proj_body(x_ref, w_ref, b_ref, zn_ref):
    z = lax.dot_general(
        x_ref[...], w_ref[...], (((1,), (1,)), ((), ())),
        preferred_element_type=jnp.float32,
        precision=lax.Precision.DEFAULT,
    ) + b_ref[...]
    nrm = jnp.sqrt(jnp.sum(z * z, axis=1, keepdims=True))
    zn_ref[...] = z * (SCALE / jnp.maximum(nrm, 1e-12))


def _proj(x, W, b2, bn=2000):
    return pl.pallas_call(
        _proj_body,
        grid=(N // bn,),
        in_specs=[
            pl.BlockSpec((bn, D), lambda i: (i, 0)),
            pl.BlockSpec((D, D), lambda i: (0, 0)),
            pl.BlockSpec((1, D), lambda i: (0, 0)),
        ],
        out_specs=pl.BlockSpec((bn, D), lambda i: (i, 0)),
        out_shape=jax.ShapeDtypeStruct((N, D), jnp.float32),
    )(x, W, b2)


def _scale_body(zn_ref, ds_ref, zs_ref, dinv_ref):
    i = pl.program_id(0)
    rows = lax.broadcasted_iota(jnp.int32, (ZS_BN, 1), 0) + i * ZS_BN
    real = rows < N
    deg = ds_ref[...] + 1.0 - jnp.where(rows < N_PAD, 1.0, 0.0)
    dinv = jnp.where(real, lax.rsqrt(deg), 0.0)
    zs_ref[...] = jnp.where(real, zn_ref[...] * dinv, 0.0)
    dinv_ref[...] = dinv


def _scale(zn, dsum):
    return pl.pallas_call(
        _scale_body,
        grid=(N_ZS // ZS_BN,),
        in_specs=[
            pl.BlockSpec((ZS_BN, D), lambda i: (i, 0)),
            pl.BlockSpec((ZS_BN, 1), lambda i: (i, 0)),
        ],
        out_specs=[
            pl.BlockSpec((ZS_BN, D), lambda i: (i, 0)),
            pl.BlockSpec((ZS_BN, 1), lambda i: (i, 0)),
        ],
        out_shape=[
            jax.ShapeDtypeStruct((N_ZS, D), jnp.float32),
            jax.ShapeDtypeStruct((N_ZS, 1), jnp.float32),
        ],
    )(zn, dsum)


def _comb_body(p_ref, zs_ref, dinv_ref, out_ref):
    out_ref[...] = dinv_ref[...] * (p_ref[0] + p_ref[1] - zs_ref[...])


def _combine(parts, zs, dinv, bn=ZS_BN):
    return pl.pallas_call(
        _comb_body,
        grid=(N_ZS // bn,),
        in_specs=[
            pl.BlockSpec((NC, bn, D), lambda i: (0, i, 0)),
            pl.BlockSpec((bn, D), lambda i: (i, 0)),
            pl.BlockSpec((bn, 1), lambda i: (i, 0)),
        ],
        out_specs=pl.BlockSpec((bn, D), lambda i: (i, 0)),
        out_shape=jax.ShapeDtypeStruct((N, D), jnp.float32),
    )(parts, zs, dinv)


@jax.jit
def kernel(x, edge_index, W, b):
    src1, dst1 = _eprep(edge_index.astype(jnp.int32))

    deg_p = _deg_kernel()(dst1)
    zn = _proj(x, W, b.reshape(1, D))

    dsum = (deg_p[0] + deg_p[1])[:N].reshape(N, 1)
    zs, dinv = _scale(zn, dsum)

    parts = _msg_kernel()(zs, src1, dst1)
    return _combine(parts, zs, dinv)
